# baseline (speedup 1.0000x reference)
import jax
import jax.numpy as jnp
from jax import lax
from jax.experimental import pallas as pl
from jax.experimental.pallas import tpu as pltpu

NT = 8192
NKV = NT // 2
NF_IN = 128
NF = 256
EPS = 1e-4

BL = 512
NBL = NT // BL
BKV = NKV // NBL
BA = 128
NBA = NT // BA


def _bf(a):
    return a.astype(jnp.bfloat16)


def _dot(a, b):
    return jnp.dot(_bf(a), _bf(b), preferred_element_type=jnp.float32)


def _affine(st, n, g, b):
    s0, s1 = st
    mu = s0 / n
    var = s1 / n - mu * mu
    a = g * lax.rsqrt(var + EPS)
    c = b - mu * a
    return a, c


def _zst():
    return (jnp.zeros((1, NF), jnp.float32), jnp.zeros((1, NF), jnp.float32))


def _acc(st, yb):
    return (st[0] + jnp.sum(yb, axis=0, keepdims=True),
            st[1] + jnp.sum(yb * yb, axis=0, keepdims=True))


def _mega(flat, flat_even, wp1, wkv, wq, wk, wv, wtrans, wr1, wr2, gb,
          out, X, KVP, K, V, E, F, SA, SB):
    def gbrow(r):
        return gb[r:r + 1, :]

    def p1(i, st):
        r = pl.ds(i * BL, BL)
        xb = _dot(flat[r, :], wp1[...])
        X[r, :] = xb
        kb = _dot(_dot(flat_even[pl.ds(i * BKV, BKV), :], wp1[...]),
                  wkv[...])
        KVP[pl.ds(i * BKV, BKV), :] = kb
        return _acc(st, kb)

    st_kv = lax.fori_loop(0, NBL, p1, _zst())
    a_kv, c_kv = _affine(st_kv, NKV, gbrow(0), gbrow(1))

    def p2(i, st):
        r = pl.ds(i * BL, BL)
        kn = KVP[r, :] * a_kv + c_kv
        K[r, :] = _bf(_dot(kn, wk[...]))
        V[r, :] = _bf(_dot(kn, wv[...]))
        return st

    lax.fori_loop(0, NKV // BL, p2, 0)

    def _scores(S, b):
        r = pl.ds(b * BA, BA)
        qb = _bf(_dot(X[r, :], wq[...]))
        S[...] = lax.dot_general(qb, K[...], (((1,), (1,)), ((), ())),
                                 preferred_element_type=jnp.float32)

    def _consume(S, b, st):
        s = S[...]
        m = jnp.max(s, axis=1, keepdims=True)
        p = jnp.exp(s - m)
        d = jnp.sum(p, axis=1, keepdims=True)
        xr = jnp.dot(_bf(p), V[...], preferred_element_type=jnp.float32) / d
        yb = _dot(xr, wtrans[...])
        E[pl.ds(b * BA, BA), :] = yb
        return _acc(st, yb)

    _scores(SA, 0)

    def p3(io, st):
        b0 = 2 * io
        _scores(SB, b0 + 1)
        st = _consume(SA, b0, st)
        _scores(SA, jnp.where(io == NBA // 2 - 1, 0, b0 + 2))
        st = _consume(SB, b0 + 1, st)
        return st

    st_y = lax.fori_loop(0, NBA // 2, p3, _zst())
    a_an, c_an = _affine(st_y, NT, gbrow(2), gbrow(3))

    def p4(i, st):
        r = pl.ds(i * BL, BL)
        x2 = X[r, :] + E[r, :] * a_an + c_an
        F[r, :] = x2
        return _acc(st, x2)

    st_x2 = lax.fori_loop(0, NBL, p4, _zst())
    a1, c1 = _affine(st_x2, NT, gbrow(4), gbrow(5))

    def p5(i, st):
        r = pl.ds(i * BL, BL)
        h1 = _dot(jnp.maximum(F[r, :] * a1 + c1, 0.0), wr1[...])
        E[r, :] = h1
        return _acc(st, h1)

    st_h1 = lax.fori_loop(0, NBL, p5, _zst())
    a2, c2 = _affine(st_h1, NT, gbrow(6), gbrow(7))

    def p6(i, st):
        r = pl.ds(i * BL, BL)
        x3 = F[r, :] + _dot(jnp.maximum(E[r, :] * a2 + c2, 0.0), wr2[...])
        out[r, :] = x3
        return _acc(st, x3)

    st_x3 = lax.fori_loop(0, NBL, p6, _zst())
    a3, c3 = _affine(st_x3, NT, gbrow(8), gbrow(9))

    def p7(i, st):
        r = pl.ds(i * BL, BL)
        out[r, :] = jnp.maximum(out[r, :] * a3 + c3, 0.0)
        return st

    lax.fori_loop(0, NBL, p7, 0)


def kernel(flat, cu_seqlens, W_p1, W_kv, g_kv, b_kv, W_q, W_k, W_v,
           W_trans, g_an, b_an, g1, b1, W_r1, g2, b2, W_r2, g3, b3):
    del cu_seqlens
    f32 = jnp.float32
    bf16 = jnp.bfloat16
    flat_even = flat[::2]
    gb = jnp.stack([g_kv, b_kv, g_an, b_an, g1, b1, g2, b2, g3, b3])

    return pl.pallas_call(
        _mega,
        grid=(1,),
        out_shape=jax.ShapeDtypeStruct((NT, NF), f32),
        scratch_shapes=[
            pltpu.VMEM((NT, NF), f32),
            pltpu.VMEM((NKV, NF), f32),
            pltpu.VMEM((NKV, NF), bf16),
            pltpu.VMEM((NKV, NF), bf16),
            pltpu.VMEM((NT, NF), f32),
            pltpu.VMEM((NT, NF), f32),
            pltpu.VMEM((BA, NKV), f32),
            pltpu.VMEM((BA, NKV), f32),
        ],
    )(flat, flat_even, W_p1, W_kv, W_q, W_k, W_v, W_trans, W_r1, W_r2, gb)


# speedup vs baseline: 1.0893x; 1.0893x over previous
import jax
import jax.numpy as jnp
from jax import lax
from jax.experimental import pallas as pl
from jax.experimental.pallas import tpu as pltpu

NT = 8192
NKV = NT // 2
NF_IN = 128
NF = 256
EPS = 1e-4

BL = 512
NBL = NT // BL
BKV = NKV // NBL
BA = 128
NBA = NT // BA


def _bf(a):
    return a.astype(jnp.bfloat16)


def _dot(a, b):
    return jnp.dot(_bf(a), _bf(b), preferred_element_type=jnp.float32)


def _affine(st, n, g, b):
    s0, s1 = st
    mu = s0 / n
    var = s1 / n - mu * mu
    a = g * lax.rsqrt(var + EPS)
    c = b - mu * a
    return a, c


def _zst():
    return (jnp.zeros((1, NF), jnp.float32), jnp.zeros((1, NF), jnp.float32))


def _acc(st, yb):
    return (st[0] + jnp.sum(yb, axis=0, keepdims=True),
            st[1] + jnp.sum(yb * yb, axis=0, keepdims=True))


def _mega(flat, flat_even, wp1, wkv, wq, wk, wv, wtrans, wr1, wr2, gb,
          out, X, KVP, K, V, E, F):
    def gbrow(r):
        return gb[r:r + 1, :]

    def p1(i, st):
        r = pl.ds(i * BL, BL)
        xb = _dot(flat[r, :], wp1[...])
        X[r, :] = xb
        kb = _dot(_dot(flat_even[pl.ds(i * BKV, BKV), :], wp1[...]),
                  wkv[...])
        KVP[pl.ds(i * BKV, BKV), :] = kb
        return _acc(st, kb)

    st_kv = lax.fori_loop(0, NBL, p1, _zst())
    a_kv, c_kv = _affine(st_kv, NKV, gbrow(0), gbrow(1))

    def p2(i, st):
        r = pl.ds(i * BL, BL)
        kn = KVP[r, :] * a_kv + c_kv
        K[r, :] = _bf(_dot(kn, wk[...]))
        V[r, :] = _bf(_dot(kn, wv[...]))
        return st

    lax.fori_loop(0, NKV // BL, p2, 0)

    def _scores(b):
        qb = _bf(_dot(X[pl.ds(b * BA, BA), :], wq[...]))
        return lax.dot_general(qb, K[...], (((1,), (1,)), ((), ())),
                               preferred_element_type=jnp.float32)

    def _consume(s, b, st):
        m = jnp.max(s, axis=1, keepdims=True)
        p = jnp.exp(s - m)
        d = jnp.sum(p, axis=1, keepdims=True)
        xr = jnp.dot(_bf(p), V[...], preferred_element_type=jnp.float32) / d
        yb = _dot(xr, wtrans[...])
        E[pl.ds(b * BA, BA), :] = yb
        return _acc(st, yb)

    def p3(io, st):
        b0 = 2 * io
        s0 = _scores(b0)
        s1 = _scores(b0 + 1)
        st = _consume(s0, b0, st)
        st = _consume(s1, b0 + 1, st)
        return st

    st_y = lax.fori_loop(0, NBA // 2, p3, _zst())
    a_an, c_an = _affine(st_y, NT, gbrow(2), gbrow(3))

    def p4(i, st):
        r = pl.ds(i * BL, BL)
        x2 = X[r, :] + E[r, :] * a_an + c_an
        F[r, :] = x2
        return _acc(st, x2)

    st_x2 = lax.fori_loop(0, NBL, p4, _zst())
    a1, c1 = _affine(st_x2, NT, gbrow(4), gbrow(5))

    def p5(i, st):
        r = pl.ds(i * BL, BL)
        h1 = _dot(jnp.maximum(F[r, :] * a1 + c1, 0.0), wr1[...])
        E[r, :] = h1
        return _acc(st, h1)

    st_h1 = lax.fori_loop(0, NBL, p5, _zst())
    a2, c2 = _affine(st_h1, NT, gbrow(6), gbrow(7))

    def p6(i, st):
        r = pl.ds(i * BL, BL)
        x3 = F[r, :] + _dot(jnp.maximum(E[r, :] * a2 + c2, 0.0), wr2[...])
        out[r, :] = x3
        return _acc(st, x3)

    st_x3 = lax.fori_loop(0, NBL, p6, _zst())
    a3, c3 = _affine(st_x3, NT, gbrow(8), gbrow(9))

    def p7(i, st):
        r = pl.ds(i * BL, BL)
        out[r, :] = jnp.maximum(out[r, :] * a3 + c3, 0.0)
        return st

    lax.fori_loop(0, NBL, p7, 0)


def kernel(flat, cu_seqlens, W_p1, W_kv, g_kv, b_kv, W_q, W_k, W_v,
           W_trans, g_an, b_an, g1, b1, W_r1, g2, b2, W_r2, g3, b3):
    del cu_seqlens
    f32 = jnp.float32
    bf16 = jnp.bfloat16
    flat_even = flat[::2]
    gb = jnp.stack([g_kv, b_kv, g_an, b_an, g1, b1, g2, b2, g3, b3])

    return pl.pallas_call(
        _mega,
        grid=(1,),
        out_shape=jax.ShapeDtypeStruct((NT, NF), f32),
        scratch_shapes=[
            pltpu.VMEM((NT, NF), f32),
            pltpu.VMEM((NKV, NF), f32),
            pltpu.VMEM((NKV, NF), bf16),
            pltpu.VMEM((NKV, NF), bf16),
            pltpu.VMEM((NT, NF), f32),
            pltpu.VMEM((NT, NF), f32),
        ],
    )(flat, flat_even, W_p1, W_kv, W_q, W_k, W_v, W_trans, W_r1, W_r2, gb)


# speedup vs baseline: 1.1918x; 1.0941x over previous
import jax
import jax.numpy as jnp
from jax import lax
from jax.experimental import pallas as pl
from jax.experimental.pallas import tpu as pltpu

NT = 8192
NKV = NT // 2
NF_IN = 128
NF = 256
EPS = 1e-4

BL = 512
NBL = NT // BL
BKV = NKV // NBL
BA = 256
NBA = NT // BA


def _bf(a):
    return a.astype(jnp.bfloat16)


def _dot(a, b):
    return jnp.dot(_bf(a), _bf(b), preferred_element_type=jnp.float32)


def _affine(st, n, g, b):
    s0, s1 = st
    mu = s0 / n
    var = s1 / n - mu * mu
    a = g * lax.rsqrt(var + EPS)
    c = b - mu * a
    return a, c


def _zst():
    return (jnp.zeros((1, NF), jnp.float32), jnp.zeros((1, NF), jnp.float32))


def _acc(st, yb):
    return (st[0] + jnp.sum(yb, axis=0, keepdims=True),
            st[1] + jnp.sum(yb * yb, axis=0, keepdims=True))


def _mega(flat, flat_even, wp1, wkv, wq, wk, wv, wtrans, wr1, wr2, gb,
          out, X, KVP, K, V, E, F):
    def gbrow(r):
        return gb[r:r + 1, :]

    def p1(i, st):
        r = pl.ds(i * BL, BL)
        xb = _dot(flat[r, :], wp1[...])
        X[r, :] = xb
        kb = _dot(_dot(flat_even[pl.ds(i * BKV, BKV), :], wp1[...]),
                  wkv[...])
        KVP[pl.ds(i * BKV, BKV), :] = kb
        return _acc(st, kb)

    st_kv = lax.fori_loop(0, NBL, p1, _zst())
    a_kv, c_kv = _affine(st_kv, NKV, gbrow(0), gbrow(1))

    def p2(i, st):
        r = pl.ds(i * BL, BL)
        kn = KVP[r, :] * a_kv + c_kv
        K[r, :] = _bf(_dot(kn, wk[...]))
        V[r, :] = _bf(_dot(kn, wv[...]))
        return st

    lax.fori_loop(0, NKV // BL, p2, 0)

    def p3(i, st):
        r = pl.ds(i * BA, BA)
        xb = X[r, :]
        qb = _bf(_dot(xb, wq[...]))
        s = lax.dot_general(qb, K[...], (((1,), (1,)), ((), ())),
                            preferred_element_type=jnp.float32)
        m = jnp.max(s, axis=1, keepdims=True)
        p = jnp.exp(s - m)
        d = jnp.sum(p, axis=1, keepdims=True)
        xr = jnp.dot(_bf(p), V[...], preferred_element_type=jnp.float32) / d
        yb = _dot(xr, wtrans[...])
        E[r, :] = yb
        sy, syy = st[0]
        sx, sxx, sxy = st[1]
        return ((sy + jnp.sum(yb, axis=0, keepdims=True),
                 syy + jnp.sum(yb * yb, axis=0, keepdims=True)),
                (sx + jnp.sum(xb, axis=0, keepdims=True),
                 sxx + jnp.sum(xb * xb, axis=0, keepdims=True),
                 sxy + jnp.sum(xb * yb, axis=0, keepdims=True)))

    z = jnp.zeros((1, NF), jnp.float32)
    (st_y, st_x) = lax.fori_loop(0, NBA, p3, ((z, z), (z, z, z)))
    a_an, c_an = _affine(st_y, NT, gbrow(2), gbrow(3))

    sx, sxx, sxy = st_x
    sy, syy = st_y
    s2_0 = sx + a_an * sy + NT * c_an
    s2_1 = (sxx + 2.0 * a_an * sxy + 2.0 * c_an * sx + a_an * a_an * syy
            + 2.0 * a_an * c_an * sy + NT * c_an * c_an)
    a1, c1 = _affine((s2_0, s2_1), NT, gbrow(4), gbrow(5))

    def p5(i, st):
        r = pl.ds(i * BL, BL)
        x2 = X[r, :] + E[r, :] * a_an + c_an
        F[r, :] = x2
        h1 = _dot(jnp.maximum(x2 * a1 + c1, 0.0), wr1[...])
        E[r, :] = h1
        return _acc(st, h1)

    st_h1 = lax.fori_loop(0, NBL, p5, _zst())
    a2, c2 = _affine(st_h1, NT, gbrow(6), gbrow(7))

    def p6(i, st):
        r = pl.ds(i * BL, BL)
        x3 = F[r, :] + _dot(jnp.maximum(E[r, :] * a2 + c2, 0.0), wr2[...])
        out[r, :] = x3
        return _acc(st, x3)

    st_x3 = lax.fori_loop(0, NBL, p6, _zst())
    a3, c3 = _affine(st_x3, NT, gbrow(8), gbrow(9))

    def p7(i, st):
        r = pl.ds(i * BL, BL)
        out[r, :] = jnp.maximum(out[r, :] * a3 + c3, 0.0)
        return st

    lax.fori_loop(0, NBL, p7, 0)


def kernel(flat, cu_seqlens, W_p1, W_kv, g_kv, b_kv, W_q, W_k, W_v,
           W_trans, g_an, b_an, g1, b1, W_r1, g2, b2, W_r2, g3, b3):
    del cu_seqlens
    f32 = jnp.float32
    bf16 = jnp.bfloat16
    flat_even = flat[::2]
    gb = jnp.stack([g_kv, b_kv, g_an, b_an, g1, b1, g2, b2, g3, b3])

    return pl.pallas_call(
        _mega,
        grid=(1,),
        out_shape=jax.ShapeDtypeStruct((NT, NF), f32),
        scratch_shapes=[
            pltpu.VMEM((NT, NF), f32),
            pltpu.VMEM((NKV, NF), f32),
            pltpu.VMEM((NKV, NF), bf16),
            pltpu.VMEM((NKV, NF), bf16),
            pltpu.VMEM((NT, NF), f32),
            pltpu.VMEM((NT, NF), f32),
        ],
    )(flat, flat_even, W_p1, W_kv, W_q, W_k, W_v, W_trans, W_r1, W_r2, gb)


# speedup vs baseline: 1.1952x; 1.0029x over previous
import jax
import jax.numpy as jnp
from jax import lax
from jax.experimental import pallas as pl
from jax.experimental.pallas import tpu as pltpu

NT = 8192
NKV = NT // 2
NF_IN = 128
NF = 256
EPS = 1e-4

BL = 512
NBL = NT // BL
BKV = NKV // NBL
BL2 = 1024
NBL2 = NT // BL2
BA = 256
NBA = NT // BA


def _bf(a):
    return a.astype(jnp.bfloat16)


def _dot(a, b):
    return jnp.dot(_bf(a), _bf(b), preferred_element_type=jnp.float32)


def _affine(st, n, g, b):
    s0, s1 = st
    mu = s0 / n
    var = s1 / n - mu * mu
    a = g * lax.rsqrt(var + EPS)
    c = b - mu * a
    return a, c


def _zst():
    return (jnp.zeros((1, NF), jnp.float32), jnp.zeros((1, NF), jnp.float32))


def _acc(st, yb):
    return (st[0] + jnp.sum(yb, axis=0, keepdims=True),
            st[1] + jnp.sum(yb * yb, axis=0, keepdims=True))


def _mega(flat, flat_even, wp1, wkv, wq, wk, wv, wtrans, wr1, wr2, gb,
          out, X, Q, K, V, E, F):
    def gbrow(r):
        return gb[r:r + 1, :]

    def p1(i, st):
        r = pl.ds(i * BL, BL)
        xb = _dot(flat[r, :], wp1[...])
        X[r, :] = xb
        Q[r, :] = _bf(_dot(xb, wq[...]))
        kb = _dot(_dot(flat_even[pl.ds(i * BKV, BKV), :], wp1[...]),
                  wkv[...])
        return _acc(st, kb)

    st_kv = lax.fori_loop(0, NBL, p1, _zst())
    a_kv, c_kv = _affine(st_kv, NKV, gbrow(0), gbrow(1))

    def p2(i, st):
        r = pl.ds(i * BL, BL)
        kb = _dot(_dot(flat_even[r, :], wp1[...]), wkv[...])
        kn = kb * a_kv + c_kv
        K[r, :] = _bf(_dot(kn, wk[...]))
        V[r, :] = _bf(_dot(kn, wv[...]))
        return st

    lax.fori_loop(0, NKV // BL, p2, 0)

    def p3(i, st):
        r = pl.ds(i * BA, BA)
        xb = X[r, :]
        s = lax.dot_general(Q[r, :], K[...], (((1,), (1,)), ((), ())),
                            preferred_element_type=jnp.float32)
        m = jnp.max(s, axis=1, keepdims=True)
        p = jnp.exp(s - m)
        d = jnp.sum(p, axis=1, keepdims=True)
        xr = jnp.dot(_bf(p), V[...], preferred_element_type=jnp.float32) / d
        yb = _dot(xr, wtrans[...])
        E[r, :] = yb
        sy, syy = st[0]
        sx, sxx, sxy = st[1]
        return ((sy + jnp.sum(yb, axis=0, keepdims=True),
                 syy + jnp.sum(yb * yb, axis=0, keepdims=True)),
                (sx + jnp.sum(xb, axis=0, keepdims=True),
                 sxx + jnp.sum(xb * xb, axis=0, keepdims=True),
                 sxy + jnp.sum(xb * yb, axis=0, keepdims=True)))

    z = jnp.zeros((1, NF), jnp.float32)
    (st_y, st_x) = lax.fori_loop(0, NBA, p3, ((z, z), (z, z, z)))
    a_an, c_an = _affine(st_y, NT, gbrow(2), gbrow(3))

    sx, sxx, sxy = st_x
    sy, syy = st_y
    s2_0 = sx + a_an * sy + NT * c_an
    s2_1 = (sxx + 2.0 * a_an * sxy + 2.0 * c_an * sx + a_an * a_an * syy
            + 2.0 * a_an * c_an * sy + NT * c_an * c_an)
    a1, c1 = _affine((s2_0, s2_1), NT, gbrow(4), gbrow(5))

    def p5(i, st):
        r = pl.ds(i * BL2, BL2)
        x2 = X[r, :] + E[r, :] * a_an + c_an
        F[r, :] = x2
        h1 = _dot(jnp.maximum(x2 * a1 + c1, 0.0), wr1[...])
        E[r, :] = h1
        return _acc(st, h1)

    st_h1 = lax.fori_loop(0, NBL2, p5, _zst())
    a2, c2 = _affine(st_h1, NT, gbrow(6), gbrow(7))

    def p6(i, st):
        r = pl.ds(i * BL2, BL2)
        x3 = F[r, :] + _dot(jnp.maximum(E[r, :] * a2 + c2, 0.0), wr2[...])
        out[r, :] = x3
        return _acc(st, x3)

    st_x3 = lax.fori_loop(0, NBL2, p6, _zst())
    a3, c3 = _affine(st_x3, NT, gbrow(8), gbrow(9))

    def p7(i, st):
        r = pl.ds(i * BL2, BL2)
        out[r, :] = jnp.maximum(out[r, :] * a3 + c3, 0.0)
        return st

    lax.fori_loop(0, NBL2, p7, 0)


def kernel(flat, cu_seqlens, W_p1, W_kv, g_kv, b_kv, W_q, W_k, W_v,
           W_trans, g_an, b_an, g1, b1, W_r1, g2, b2, W_r2, g3, b3):
    del cu_seqlens
    f32 = jnp.float32
    bf16 = jnp.bfloat16
    flat_even = flat[::2]
    gb = jnp.stack([g_kv, b_kv, g_an, b_an, g1, b1, g2, b2, g3, b3])

    return pl.pallas_call(
        _mega,
        grid=(1,),
        out_shape=jax.ShapeDtypeStruct((NT, NF), f32),
        scratch_shapes=[
            pltpu.VMEM((NT, NF), f32),
            pltpu.VMEM((NT, NF), bf16),
            pltpu.VMEM((NKV, NF), bf16),
            pltpu.VMEM((NKV, NF), bf16),
            pltpu.VMEM((NT, NF), f32),
            pltpu.VMEM((NT, NF), f32),
        ],
    )(flat, flat_even, W_p1, W_kv, W_q, W_k, W_v, W_trans, W_r1, W_r2, gb)


# speedup vs baseline: 1.2318x; 1.0306x over previous
import jax
import jax.numpy as jnp
from jax import lax
from jax.experimental import pallas as pl
from jax.experimental.pallas import tpu as pltpu

NT = 8192
NKV = NT // 2
NF_IN = 128
NF = 256
EPS = 1e-4

BL = 512
NBL = NT // BL
BKV = NKV // NBL
BL2 = 1024
NBL2 = NT // BL2
BA = 256
NBA = NT // BA


def _bf(a):
    return a.astype(jnp.bfloat16)


def _dot(a, b):
    return jnp.dot(_bf(a), _bf(b), preferred_element_type=jnp.float32)


def _affine(st, n, g, b):
    s0, s1 = st
    mu = s0 / n
    var = s1 / n - mu * mu
    a = g * lax.rsqrt(var + EPS)
    c = b - mu * a
    return a, c


def _zst():
    return (jnp.zeros((1, NF), jnp.float32), jnp.zeros((1, NF), jnp.float32))


def _acc(st, yb):
    return (st[0] + jnp.sum(yb, axis=0, keepdims=True),
            st[1] + jnp.sum(yb * yb, axis=0, keepdims=True))


def _mega(flat, flat_even, wp1, wkv, wq, wk, wv, wtrans, wr1, wr2, gb,
          out, X, Q, K, V, E, F, PB):
    def gbrow(r):
        return gb[r:r + 1, :]

    def p1(i, st):
        r = pl.ds(i * BL, BL)
        xb = _dot(flat[r, :], wp1[...])
        X[r, :] = xb
        Q[r, :] = _bf(_dot(xb, wq[...]))
        kb = _dot(_dot(flat_even[pl.ds(i * BKV, BKV), :], wp1[...]),
                  wkv[...])
        return _acc(st, kb)

    st_kv = lax.fori_loop(0, NBL, p1, _zst())
    a_kv, c_kv = _affine(st_kv, NKV, gbrow(0), gbrow(1))

    def p2(i, st):
        r = pl.ds(i * BL, BL)
        kb = _dot(_dot(flat_even[r, :], wp1[...]), wkv[...])
        kn = kb * a_kv + c_kv
        K[r, :] = _bf(_dot(kn, wk[...]))
        V[r, :] = _bf(_dot(kn, wv[...]))
        return st

    lax.fori_loop(0, NKV // BL, p2, 0)

    CW = 512

    def p3(i, st):
        r = pl.ds(i * BA, BA)
        xb = X[r, :]
        s = lax.dot_general(Q[r, :], K[...], (((1,), (1,)), ((), ())),
                            preferred_element_type=jnp.float32)
        m = jnp.max(s, axis=1, keepdims=True)

        d = jnp.zeros((BA, 1), jnp.float32)
        for j in range(NKV // CW):
            pj = jnp.exp(s[:, j * CW:(j + 1) * CW] - m)
            PB[:, j * CW:(j + 1) * CW] = _bf(pj)
            d = d + jnp.sum(pj, axis=1, keepdims=True)
        xr = jnp.dot(PB[...], V[...], preferred_element_type=jnp.float32) / d
        yb = _dot(xr, wtrans[...])
        E[r, :] = yb
        sy, syy = st[0]
        sx, sxx, sxy = st[1]
        return ((sy + jnp.sum(yb, axis=0, keepdims=True),
                 syy + jnp.sum(yb * yb, axis=0, keepdims=True)),
                (sx + jnp.sum(xb, axis=0, keepdims=True),
                 sxx + jnp.sum(xb * xb, axis=0, keepdims=True),
                 sxy + jnp.sum(xb * yb, axis=0, keepdims=True)))

    z = jnp.zeros((1, NF), jnp.float32)
    (st_y, st_x) = lax.fori_loop(0, NBA, p3, ((z, z), (z, z, z)))
    a_an, c_an = _affine(st_y, NT, gbrow(2), gbrow(3))

    sx, sxx, sxy = st_x
    sy, syy = st_y
    s2_0 = sx + a_an * sy + NT * c_an
    s2_1 = (sxx + 2.0 * a_an * sxy + 2.0 * c_an * sx + a_an * a_an * syy
            + 2.0 * a_an * c_an * sy + NT * c_an * c_an)
    a1, c1 = _affine((s2_0, s2_1), NT, gbrow(4), gbrow(5))

    def p5(i, st):
        r = pl.ds(i * BL2, BL2)
        x2 = X[r, :] + E[r, :] * a_an + c_an
        F[r, :] = x2
        h1 = _dot(jnp.maximum(x2 * a1 + c1, 0.0), wr1[...])
        E[r, :] = h1
        return _acc(st, h1)

    st_h1 = lax.fori_loop(0, NBL2, p5, _zst())
    a2, c2 = _affine(st_h1, NT, gbrow(6), gbrow(7))

    def p6(i, st):
        r = pl.ds(i * BL2, BL2)
        x3 = F[r, :] + _dot(jnp.maximum(E[r, :] * a2 + c2, 0.0), wr2[...])
        out[r, :] = x3
        return _acc(st, x3)

    st_x3 = lax.fori_loop(0, NBL2, p6, _zst())
    a3, c3 = _affine(st_x3, NT, gbrow(8), gbrow(9))

    def p7(i, st):
        r = pl.ds(i * BL2, BL2)
        out[r, :] = jnp.maximum(out[r, :] * a3 + c3, 0.0)
        return st

    lax.fori_loop(0, NBL2, p7, 0)


def kernel(flat, cu_seqlens, W_p1, W_kv, g_kv, b_kv, W_q, W_k, W_v,
           W_trans, g_an, b_an, g1, b1, W_r1, g2, b2, W_r2, g3, b3):
    del cu_seqlens
    f32 = jnp.float32
    bf16 = jnp.bfloat16
    flat_even = flat[::2]
    gb = jnp.stack([g_kv, b_kv, g_an, b_an, g1, b1, g2, b2, g3, b3])

    return pl.pallas_call(
        _mega,
        grid=(1,),
        out_shape=jax.ShapeDtypeStruct((NT, NF), f32),
        scratch_shapes=[
            pltpu.VMEM((NT, NF), f32),
            pltpu.VMEM((NT, NF), bf16),
            pltpu.VMEM((NKV, NF), bf16),
            pltpu.VMEM((NKV, NF), bf16),
            pltpu.VMEM((NT, NF), f32),
            pltpu.VMEM((NT, NF), f32),
            pltpu.VMEM((BA, NKV), bf16),
        ],
    )(flat, flat_even, W_p1, W_kv, W_q, W_k, W_v, W_trans, W_r1, W_r2, gb)
